# Initial kernel scaffold; baseline (speedup 1.0000x reference)
import jax
import jax.numpy as jnp
from jax.experimental import pallas as pl


def kernel(x, table, W, b):
    raise NotImplementedError("write your pallas kernel here")


# speedup vs baseline: 31.2929x; 31.2929x over previous
import functools

import jax
import jax.numpy as jnp
from jax import lax
from jax.experimental import pallas as pl
from jax.experimental.pallas import tpu as pltpu
from jax.experimental.pallas import tpu_sc as plsc

VOCAB = 1000000
EMBED = 32
INPUT_SIZE = 50
BATCH = 16384

NUM_CORES = 2
NUM_SUBCORES = 16
NW = NUM_CORES * NUM_SUBCORES
ROWS_PER_W = BATCH // NW
CHUNK = 64
NCHUNKS = ROWS_PER_W // CHUNK
IDX_PER_CHUNK = CHUNK * INPUT_SIZE
GATHER_ROWS = 128
NGATHER = IDX_PER_CHUNK // GATHER_ROWS
FLAT = INPUT_SIZE * EMBED


def _disc_body(x_hbm, tab_hbm, w_hbm, b_hbm, part_hbm,
               idx_v, rows_v, wv, bv, partials, sem):
    cid = lax.axis_index("c")
    sid = lax.axis_index("s")
    wid = sid * NUM_CORES + cid

    pltpu.sync_copy(w_hbm, wv)
    pltpu.sync_copy(b_hbm, bv)

    def chunk_body(c, carry):
        row0 = wid * ROWS_PER_W + c * CHUNK
        xoff = wid * (NCHUNKS * IDX_PER_CHUNK) + c * IDX_PER_CHUNK

        pltpu.sync_copy(x_hbm.at[pl.ds(xoff, IDX_PER_CHUNK)], idx_v)
        descs = [
            pltpu.async_copy(
                tab_hbm.at[idx_v.at[pl.ds(i * GATHER_ROWS, GATHER_ROWS)]],
                rows_v.at[pl.ds(i * GATHER_ROWS, GATHER_ROWS)],
                sem,
            )
            for i in range(NGATHER)
        ]
        for d in descs:
            d.wait()

        bias = bv[...]

        def row_body(r, rcarry):
            g0 = r * INPUT_SIZE

            def j_body(j, acc):
                a0, a1 = acc
                g = g0 + j
                a0 = a0 + rows_v[g, pl.ds(0, 16)] * wv[pl.ds(j * 32, 16)]
                a1 = a1 + rows_v[g, pl.ds(16, 16)] * wv[pl.ds(j * 32 + 16, 16)]
                return (a0, a1)

            zero = jnp.zeros((16,), jnp.float32)
            a0, a1 = lax.fori_loop(0, INPUT_SIZE, j_body, (zero, zero))
            partials[pl.ds(r * 16, 16)] = a0 + a1 + bias
            return rcarry

        lax.fori_loop(0, CHUNK, row_body, 0)

        pltpu.sync_copy(partials, part_hbm.at[pl.ds(row0 * 16, CHUNK * 16)])
        return carry

    lax.fori_loop(0, NCHUNKS, chunk_body, 0)


def _finalize_body(p_ref, o_ref):
    z = jnp.sum(p_ref[...], axis=1, keepdims=True)
    o_ref[...] = 1.0 / (1.0 + jnp.exp(-z))


def kernel(x, table, W, b):
    xf = x.astype(jnp.int32).reshape(BATCH * INPUT_SIZE)
    wf = W.reshape(FLAT).astype(jnp.float32)
    b16 = jnp.where(jnp.arange(16) == 0, b[0].astype(jnp.float32), 0.0)

    mesh = plsc.VectorSubcoreMesh(core_axis_name="c", subcore_axis_name="s")
    sc = pl.kernel(
        _disc_body,
        out_type=jax.ShapeDtypeStruct((BATCH * 16,), jnp.float32),
        mesh=mesh,
        compiler_params=pltpu.CompilerParams(use_tc_tiling_on_sc=False),
        scratch_types=[
            pltpu.VMEM((IDX_PER_CHUNK,), jnp.int32),
            pltpu.VMEM((IDX_PER_CHUNK, EMBED), jnp.float32),
            pltpu.VMEM((FLAT,), jnp.float32),
            pltpu.VMEM((16,), jnp.float32),
            pltpu.VMEM((CHUNK * 16,), jnp.float32),
            pltpu.SemaphoreType.DMA,
        ],
    )
    partials = sc(xf, table, wf, b16).reshape(BATCH, 16)

    blk = 2048
    out = pl.pallas_call(
        _finalize_body,
        out_shape=jax.ShapeDtypeStruct((BATCH, 1), jnp.float32),
        grid=(BATCH // blk,),
        in_specs=[pl.BlockSpec((blk, 16), lambda i: (i, 0))],
        out_specs=pl.BlockSpec((blk, 1), lambda i: (i, 0)),
    )(partials)
    return out


# speedup vs baseline: 35.9093x; 1.1475x over previous
import functools

import jax
import jax.numpy as jnp
from jax import lax
from jax.experimental import pallas as pl
from jax.experimental.pallas import tpu as pltpu
from jax.experimental.pallas import tpu_sc as plsc

VOCAB = 1000000
EMBED = 32
INPUT_SIZE = 50
BATCH = 16384

NUM_CORES = 2
NUM_SUBCORES = 16
NW = NUM_CORES * NUM_SUBCORES
ROWS_PER_W = BATCH // NW
CHUNK = 32
NCHUNKS = ROWS_PER_W // CHUNK
IDX_PER_CHUNK = CHUNK * INPUT_SIZE
GATHER_ROWS = 64
NGATHER = IDX_PER_CHUNK // GATHER_ROWS
FLAT = INPUT_SIZE * EMBED
RB = 8
NB = CHUNK // RB


def _disc_body(x_hbm, tab_hbm, w_hbm, b_hbm, part_hbm,
               idx_a, idx_b, buf_a, buf_b, wv, bv, partials, sem_a, sem_b):
    cid = lax.axis_index("c")
    sid = lax.axis_index("s")
    wid = sid * NUM_CORES + cid

    pltpu.sync_copy(w_hbm, wv)
    pltpu.sync_copy(b_hbm, bv)

    def fire(idx_ref, buf_ref, sem, c):
        xoff = wid * (NCHUNKS * IDX_PER_CHUNK) + c * IDX_PER_CHUNK
        pltpu.sync_copy(x_hbm.at[pl.ds(xoff, IDX_PER_CHUNK)], idx_ref)
        for i in range(NGATHER):
            pltpu.async_copy(
                tab_hbm.at[idx_ref.at[pl.ds(i * GATHER_ROWS, GATHER_ROWS)]],
                buf_ref.at[pl.ds(i * GATHER_ROWS, GATHER_ROWS)],
                sem,
            )

    def drain(buf_ref, sem):
        pltpu.make_async_copy(
            tab_hbm.at[pl.ds(0, IDX_PER_CHUNK)], buf_ref, sem
        ).wait()

    def compute(buf_ref, c):
        bias = bv[...]

        def blk_body(t, bcarry):
            r0 = t * RB

            def j_body(j, accs):
                w0 = wv[pl.ds(j * 32, 16)]
                w1 = wv[pl.ds(j * 32 + 16, 16)]
                out = []
                for rr in range(RB):
                    g = (r0 + rr) * INPUT_SIZE + j
                    out.append(accs[2 * rr] + buf_ref[g, pl.ds(0, 16)] * w0)
                    out.append(accs[2 * rr + 1] + buf_ref[g, pl.ds(16, 16)] * w1)
                return tuple(out)

            zero = jnp.zeros((16,), jnp.float32)
            accs = lax.fori_loop(0, INPUT_SIZE, j_body, (zero,) * (2 * RB))
            for rr in range(RB):
                partials[pl.ds((r0 + rr) * 16, 16)] = (
                    accs[2 * rr] + accs[2 * rr + 1] + bias
                )
            return bcarry

        lax.fori_loop(0, NB, blk_body, 0)
        row0 = wid * ROWS_PER_W + c * CHUNK
        pltpu.sync_copy(partials, part_hbm.at[pl.ds(row0 * 16, CHUNK * 16)])

    fire(idx_a, buf_a, sem_a, 0)

    def m_body(m, carry):
        fire(idx_b, buf_b, sem_b, 2 * m + 1)
        drain(buf_a, sem_a)
        compute(buf_a, 2 * m)

        @pl.when(m < NCHUNKS // 2 - 1)
        def _():
            fire(idx_a, buf_a, sem_a, 2 * m + 2)

        drain(buf_b, sem_b)
        compute(buf_b, 2 * m + 1)
        return carry

    lax.fori_loop(0, NCHUNKS // 2, m_body, 0)


def _finalize_body(p_ref, o_ref):
    z = jnp.sum(p_ref[...], axis=1, keepdims=True)
    o_ref[...] = 1.0 / (1.0 + jnp.exp(-z))


def kernel(x, table, W, b):
    xf = x.astype(jnp.int32).reshape(BATCH * INPUT_SIZE)
    wf = W.reshape(FLAT).astype(jnp.float32)
    b16 = jnp.where(jnp.arange(16) == 0, b[0].astype(jnp.float32), 0.0)

    mesh = plsc.VectorSubcoreMesh(core_axis_name="c", subcore_axis_name="s")
    sc = pl.kernel(
        _disc_body,
        out_type=jax.ShapeDtypeStruct((BATCH * 16,), jnp.float32),
        mesh=mesh,
        compiler_params=pltpu.CompilerParams(use_tc_tiling_on_sc=False),
        scratch_types=[
            pltpu.VMEM((IDX_PER_CHUNK,), jnp.int32),
            pltpu.VMEM((IDX_PER_CHUNK,), jnp.int32),
            pltpu.VMEM((IDX_PER_CHUNK, EMBED), jnp.float32),
            pltpu.VMEM((IDX_PER_CHUNK, EMBED), jnp.float32),
            pltpu.VMEM((FLAT,), jnp.float32),
            pltpu.VMEM((16,), jnp.float32),
            pltpu.VMEM((CHUNK * 16,), jnp.float32),
            pltpu.SemaphoreType.DMA,
            pltpu.SemaphoreType.DMA,
        ],
    )
    partials = sc(xf, table, wf, b16).reshape(BATCH, 16)

    blk = 2048
    out = pl.pallas_call(
        _finalize_body,
        out_shape=jax.ShapeDtypeStruct((BATCH, 1), jnp.float32),
        grid=(BATCH // blk,),
        in_specs=[pl.BlockSpec((blk, 16), lambda i: (i, 0))],
        out_specs=pl.BlockSpec((blk, 1), lambda i: (i, 0)),
    )(partials)
    return out
